# baseline (speedup 1.0000x reference)
import functools

import jax
import jax.numpy as jnp
from jax import lax
from jax.experimental import pallas as pl
from jax.experimental.pallas import tpu as pltpu
from jax.experimental.pallas import tpu_sc as plsc

N = 10000
D = 128
DP = 160
E = 320000
NC, NS = 2, 16
NW = NC * NS
EPW = E // NW
CH = 80
NCHUNK = EPW // CH
RPT = N // NS
PROB_ATTR = 0.5
TEMP = 0.5

BR = 400
GR = N // BR
BL = 2000
GL = N // BL

_F32 = jnp.float32
_BF = jnp.bfloat16


def _sc_agg_body(xp, zrs, src, dst, w, out,
                 srcv, wv, dstage0, dstage1, rows0, rows1, acc,
                 sem0, sem1, semd0, semd1, sems0, sems1):
    c = lax.axis_index("c")
    s = lax.axis_index("s")
    base = (c * NS + s) * EPW
    pltpu.sync_copy(src.at[pl.ds(base, EPW)], srcv)
    pltpu.sync_copy(w.at[pl.ds(base, EPW)], wv)
    pltpu.sync_copy(zrs, acc.at[pl.ds(s * RPT, RPT)])
    plsc.subcore_barrier()

    rows = (rows0, rows1)
    sems = (sem0, sem1)
    dstage = (dstage0, dstage1)
    semsd = (semd0, semd1)
    semss = (sems0, sems1)

    def scale(i, rbuf):
        def scale_row(r, carry2):
            wr = plsc.load_gather(wv, [jnp.full((16,), i * CH, jnp.int32) + r])
            wrb = plsc.pack(wr, wr, format=plsc.PackFormat.INTERLEAVED)
            for k in range(D // 32):
                rbuf[r, pl.ds(k * 32, 32)] = rbuf[r, pl.ds(k * 32, 32)] * wrb
            return carry2

        lax.fori_loop(0, CH, scale_row, 0, unroll=2)

    def fetch(i, b):
        pltpu.async_copy(xp.at[srcv.at[pl.ds(i * CH, CH)]], rows[b], sems[b])
        pltpu.async_copy(dst.at[pl.ds(base + i * CH, CH)], dstage[b],
                         semsd[b])

    def await_chunk(i, b):
        pltpu.make_async_copy(xp.at[srcv.at[pl.ds(i * CH, CH)]],
                              rows[b], sems[b]).wait()
        pltpu.make_async_copy(dst.at[pl.ds(base + i * CH, CH)], dstage[b],
                              semsd[b]).wait()

    def await_scatter(b):
        pltpu.make_async_copy(rows[b], acc.at[dstage[b]], semss[b]).wait()

    fetch(0, 0)

    def pair(p, carry):
        for b in range(2):
            i = p * 2 + b
            await_chunk(i, b)

            @pl.when(i > 0)
            def _():
                await_scatter(1 - b)

            fetch(i + 1, 1 - b)
            scale(i, rows[b])
            pltpu.async_copy(rows[b], acc.at[dstage[b]], semss[b], add=True)
        return carry

    lax.fori_loop(0, NCHUNK // 2, pair, 0)
    last = NCHUNK - 1
    await_chunk(last, 0)
    await_scatter(1)
    scale(last, rows0)
    pltpu.async_copy(rows0, acc.at[dstage0], semss[0], add=True)
    await_scatter(0)
    plsc.subcore_barrier()
    pltpu.sync_copy(acc.at[pl.ds(s * RPT, RPT)],
                    out.at[pl.ds(c * N + s * RPT, RPT)])


def _sc_agg(xp, zrs, src, dst, w):
    fn = pl.kernel(
        _sc_agg_body,
        mesh=plsc.VectorSubcoreMesh(core_axis_name="c", subcore_axis_name="s",
                                    num_cores=NC, num_subcores=NS),
        out_type=jax.ShapeDtypeStruct((NC * N, DP), _BF),
        scratch_types=[
            pltpu.VMEM((EPW,), jnp.int32),
            pltpu.VMEM((EPW,), _F32),
            pltpu.VMEM((CH,), jnp.int32),
            pltpu.VMEM((CH,), jnp.int32),
            pltpu.VMEM((CH, DP), _BF),
            pltpu.VMEM((CH, DP), _BF),
            pltpu.VMEM_SHARED((N, DP), _BF),
            pltpu.SemaphoreType.DMA,
            pltpu.SemaphoreType.DMA,
            pltpu.SemaphoreType.DMA,
            pltpu.SemaphoreType.DMA,
            pltpu.SemaphoreType.DMA,
            pltpu.SemaphoreType.DMA,
        ],
        compiler_params=pltpu.CompilerParams(use_tc_tiling_on_sc=False,
                                             needs_layout_passes=False),
    )
    return fn(xp, zrs, src, dst, w)


def _prep_body(feat, mask, out):
    m = (mask[...] < PROB_ATTR).astype(_F32)
    x = feat[...] * m
    pad = jnp.concatenate(
        [jnp.ones((BR, 1), _F32), jnp.zeros((BR, DP - D - 1), _F32)], axis=1)
    out[...] = jnp.concatenate([x, pad], axis=1).astype(_BF)


def _prep(feat1, mask2):
    return pl.pallas_call(
        _prep_body,
        grid=(GR,),
        in_specs=[
            pl.BlockSpec((BR, D), lambda i: (i, 0)),
            pl.BlockSpec((1, D), lambda i: (0, 0)),
        ],
        out_specs=pl.BlockSpec((BR, DP), lambda i: (i, 0)),
        out_shape=jax.ShapeDtypeStruct((N, DP), _BF),
    )(feat1, mask2)


BS = 1000
GS = N // BS


def _sage_body(accr, xp, ws, wn, b, out, a0v, a1v, s0, s1):
    i = pl.program_id(0)
    c0 = pltpu.async_copy(accr.at[pl.ds(i * BS, BS)], a0v, s0)
    c1 = pltpu.async_copy(accr.at[pl.ds(N + i * BS, BS)], a1v, s1)
    c0.wait()
    c1.wait()
    acc = a0v[...].astype(_F32) + a1v[...].astype(_F32)
    deg = jnp.maximum(acc[:, D:D + 1], 1.0)
    agg = acc[:, :D] / deg
    x = xp[:, :D].astype(_F32)
    h = jnp.dot(x, ws[...], preferred_element_type=_F32)
    h = h + jnp.dot(agg, wn[...], preferred_element_type=_F32)
    h = jnp.maximum(h + b[...], 0.0)
    pad = jnp.concatenate(
        [jnp.ones((BS, 1), _F32), jnp.zeros((BS, DP - D - 1), _F32)], axis=1)
    out[...] = jnp.concatenate([h, pad], axis=1).astype(_BF)


def _sage(acc, xp, ws, wn, b2):
    return pl.pallas_call(
        _sage_body,
        grid=(GS,),
        in_specs=[
            pl.BlockSpec(memory_space=pltpu.HBM),
            pl.BlockSpec((BS, DP), lambda i: (i, 0)),
            pl.BlockSpec((D, D), lambda i: (0, 0)),
            pl.BlockSpec((D, D), lambda i: (0, 0)),
            pl.BlockSpec((1, D), lambda i: (0, 0)),
        ],
        out_specs=pl.BlockSpec((BS, DP), lambda i: (i, 0)),
        out_shape=jax.ShapeDtypeStruct((N, DP), _BF),
        scratch_shapes=[
            pltpu.VMEM((BS, DP), _BF),
            pltpu.VMEM((BS, DP), _BF),
            pltpu.SemaphoreType.DMA,
            pltpu.SemaphoreType.DMA,
        ],
    )(acc, xp, ws, wn, b2)


def _proj_norm(x, wpv, bpv):
    z = jnp.dot(x, wpv, preferred_element_type=_F32) + bpv
    z = jnp.where(z > 0, z, jnp.exp(jnp.minimum(z, 0.0)) - 1.0)
    n = jnp.sqrt(jnp.sum(z * z, axis=1, keepdims=True))
    return ((z / jnp.maximum(n, 1e-12)) * (TEMP ** -0.5)).astype(_BF)


def _proj_h_body(h2p, wp, bp, z1o):
    z1o[...] = _proj_norm(h2p[:, :D].astype(_F32), wp[...], bp[...])


def _proj_e_body(emb, wp, bp, z2o):
    z2o[...] = _proj_norm(emb[...], wp[...], bp[...])


def _proj(x, wp, bp2, from_padded):
    body = _proj_h_body if from_padded else _proj_e_body
    width = DP if from_padded else D
    return pl.pallas_call(
        body,
        grid=(GR,),
        in_specs=[
            pl.BlockSpec((BR, width), lambda i: (i, 0)),
            pl.BlockSpec((D, D), lambda i: (0, 0)),
            pl.BlockSpec((1, D), lambda i: (0, 0)),
        ],
        out_specs=pl.BlockSpec((BR, D), lambda i: (i, 0)),
        out_shape=jax.ShapeDtypeStruct((N, D), _BF),
    )(x, wp, bp2)


_DN = (((1,), (1,)), ((), ()))
_INV_T = 1.0 / TEMP


def _exp_sim(x, y):
    return jnp.exp(lax.dot_general(x, y, _DN, preferred_element_type=_F32))


def _loss22_body(bi_r, bj_r, r22o, d22o, r22s_, d22s_):
    i = pl.program_id(0)
    j = pl.program_id(1)
    bi = bi_r[...]
    s22 = _exp_sim(bi, bj_r[...])
    r22s = s22.sum(axis=1).reshape(1, BL)

    @pl.when(j == 0)
    def _():
        bi32 = bi.astype(_F32)
        r22s_[pl.ds(i, 1), :] = r22s
        d22s_[pl.ds(i, 1), :] = jnp.exp(jnp.sum(bi32 * bi32, axis=1)).reshape(1, BL)

    @pl.when(j > 0)
    def _():
        r22s_[pl.ds(i, 1), :] += r22s

    @pl.when((i == GL - 1) & (j == GL - 1))
    def _():
        r22o[...] = r22s_[...]
        d22o[...] = d22s_[...]


def _loss22(z2n):
    return pl.pallas_call(
        _loss22_body,
        grid=(GL, GL),
        in_specs=[
            pl.BlockSpec((BL, D), lambda i, j: (i, 0)),
            pl.BlockSpec((BL, D), lambda i, j: (j, 0)),
        ],
        out_specs=[
            pl.BlockSpec((GL, BL), lambda i, j: (0, 0)),
            pl.BlockSpec((GL, BL), lambda i, j: (0, 0)),
        ],
        out_shape=[
            jax.ShapeDtypeStruct((GL, BL), _F32),
            jax.ShapeDtypeStruct((GL, BL), _F32),
        ],
        scratch_shapes=[pltpu.VMEM((GL, BL), _F32) for _ in range(2)],
    )(z2n, z2n)


def _loss_body(ai_r, bi_r, aj_r, bj_r, r22_r, d22_r, out, r11, r12, c12,
               d11, ld12):
    i = pl.program_id(0)
    j = pl.program_id(1)
    ai = ai_r[...]
    bi = bi_r[...]
    aj = aj_r[...]
    bj = bj_r[...]

    @pl.when((i == 0) & (j == 0))
    def _():
        r11[...] = jnp.zeros_like(r11)
        r12[...] = jnp.zeros_like(r12)
        c12[...] = jnp.zeros_like(c12)

    s12 = _exp_sim(ai, bj)
    r12[pl.ds(i, 1), :] += s12.sum(axis=1).reshape(1, BL)
    c12[pl.ds(j, 1), :] += s12.sum(axis=0).reshape(1, BL)

    @pl.when(j == i)
    def _():
        s11 = _exp_sim(ai, aj)
        r11[pl.ds(i, 1), :] += s11.sum(axis=1).reshape(1, BL)

    @pl.when(j > i)
    def _():
        s11 = _exp_sim(ai, aj)
        r11[pl.ds(i, 1), :] += s11.sum(axis=1).reshape(1, BL)
        r11[pl.ds(j, 1), :] += s11.sum(axis=0).reshape(1, BL)

    @pl.when(j == 0)
    def _():
        ai32 = ai.astype(_F32)
        bi32 = bi.astype(_F32)
        d11[pl.ds(i, 1), :] = jnp.exp(jnp.sum(ai32 * ai32, axis=1)).reshape(1, BL)
        ld12[pl.ds(i, 1), :] = jnp.sum(ai32 * bi32, axis=1).reshape(1, BL)

    @pl.when((i == GL - 1) & (j == GL - 1))
    def _():
        x1 = r11[...] - d11[...] + r12[...]
        x2 = r22_r[...] - d22_r[...] + c12[...]
        l1 = jnp.log(x1) - ld12[...]
        l2 = jnp.log(x2) - ld12[...]
        out[0, 0] = jnp.sum(l1 + l2) / (2.0 * N)


def _loss(z1n, z2n, r22, d22):
    return pl.pallas_call(
        _loss_body,
        grid=(GL, GL),
        in_specs=[
            pl.BlockSpec((BL, D), lambda i, j: (i, 0)),
            pl.BlockSpec((BL, D), lambda i, j: (i, 0)),
            pl.BlockSpec((BL, D), lambda i, j: (j, 0)),
            pl.BlockSpec((BL, D), lambda i, j: (j, 0)),
            pl.BlockSpec((GL, BL), lambda i, j: (0, 0)),
            pl.BlockSpec((GL, BL), lambda i, j: (0, 0)),
        ],
        out_specs=pl.BlockSpec((1, 1), lambda i, j: (0, 0),
                               memory_space=pltpu.SMEM),
        out_shape=jax.ShapeDtypeStruct((1, 1), _F32),
        scratch_shapes=[pltpu.VMEM((GL, BL), _F32) for _ in range(5)],
    )(z1n, z2n, z1n, z2n, r22, d22)


def kernel(feat1, embds, edge_index, edge_weight, mask_rand,
           W_self0, W_neigh0, b0, W_self1, W_neigh1, b1, W_proj, b_proj):
    ei = edge_index.astype(jnp.int32)
    src = ei[0]
    dst = ei[1]
    mask2 = mask_rand.reshape(1, D)
    b0r = b0.reshape(1, D)
    b1r = b1.reshape(1, D)
    bpr = b_proj.reshape(1, D)
    zrs = jnp.zeros((RPT, DP), _BF)

    z2n = _proj(embds, W_proj, bpr, from_padded=False)
    r22, d22 = _loss22(z2n)

    xp = _prep(feat1, mask2)
    acc1 = _sc_agg(xp, zrs, src, dst, edge_weight)
    h1p = _sage(acc1, xp, W_self0, W_neigh0, b0r)
    acc2 = _sc_agg(h1p, zrs, src, dst, edge_weight)
    h2p = _sage(acc2, h1p, W_self1, W_neigh1, b1r)
    z1n = _proj(h2p, W_proj, bpr, from_padded=True)
    out = _loss(z1n, z2n, r22, d22)
    return out.reshape(())


# speedup vs baseline: 1.0178x; 1.0178x over previous
import functools

import jax
import jax.numpy as jnp
from jax import lax
from jax.experimental import pallas as pl
from jax.experimental.pallas import tpu as pltpu
from jax.experimental.pallas import tpu_sc as plsc

N = 10000
D = 128
DP = 160
E = 320000
NC, NS = 2, 16
NW = NC * NS
EPW = E // NW
CH = 80
NCHUNK = EPW // CH
RPT = N // NS
PROB_ATTR = 0.5
TEMP = 0.5

BR = 400
GR = N // BR
BL = 2000
GL = N // BL

_F32 = jnp.float32
_BF = jnp.bfloat16


def _sc_agg_body(xp, zrs, src, dst, w, out,
                 srcv, wv, dstage0, dstage1, rows0, rows1, acc,
                 sem0, sem1, semd0, semd1, sems0, sems1):
    c = lax.axis_index("c")
    s = lax.axis_index("s")
    base = (c * NS + s) * EPW
    pltpu.sync_copy(src.at[pl.ds(base, EPW)], srcv)
    pltpu.sync_copy(w.at[pl.ds(base, EPW)], wv)
    pltpu.sync_copy(zrs, acc.at[pl.ds(s * RPT, RPT)])
    plsc.subcore_barrier()

    rows = (rows0, rows1)
    sems = (sem0, sem1)
    dstage = (dstage0, dstage1)
    semsd = (semd0, semd1)
    semss = (sems0, sems1)

    def scale(i, rbuf):
        def scale_row(r, carry2):
            wr = plsc.load_gather(wv, [jnp.full((16,), i * CH, jnp.int32) + r])
            wrb = plsc.pack(wr, wr, format=plsc.PackFormat.INTERLEAVED)
            for k in range(D // 32):
                rbuf[r, pl.ds(k * 32, 32)] = rbuf[r, pl.ds(k * 32, 32)] * wrb
            return carry2

        lax.fori_loop(0, CH, scale_row, 0, unroll=2)

    def fetch(i, b):
        pltpu.async_copy(xp.at[srcv.at[pl.ds(i * CH, CH)]], rows[b], sems[b])
        pltpu.async_copy(dst.at[pl.ds(base + i * CH, CH)], dstage[b],
                         semsd[b])

    def await_chunk(i, b):
        pltpu.make_async_copy(xp.at[srcv.at[pl.ds(i * CH, CH)]],
                              rows[b], sems[b]).wait()
        pltpu.make_async_copy(dst.at[pl.ds(base + i * CH, CH)], dstage[b],
                              semsd[b]).wait()

    def await_scatter(b):
        pltpu.make_async_copy(rows[b], acc.at[dstage[b]], semss[b]).wait()

    fetch(0, 0)

    def pair(p, carry):
        for b in range(2):
            i = p * 2 + b
            await_chunk(i, b)

            @pl.when(i > 0)
            def _():
                await_scatter(1 - b)

            fetch(i + 1, 1 - b)
            scale(i, rows[b])
            pltpu.async_copy(rows[b], acc.at[dstage[b]], semss[b], add=True)
        return carry

    lax.fori_loop(0, NCHUNK // 2, pair, 0)
    last = NCHUNK - 1
    await_chunk(last, 0)
    await_scatter(1)
    scale(last, rows0)
    pltpu.async_copy(rows0, acc.at[dstage0], semss[0], add=True)
    await_scatter(0)
    plsc.subcore_barrier()
    pltpu.sync_copy(acc.at[pl.ds(s * RPT, RPT)],
                    out.at[pl.ds(c * N + s * RPT, RPT)])


def _sc_agg(xp, zrs, src, dst, w):
    fn = pl.kernel(
        _sc_agg_body,
        mesh=plsc.VectorSubcoreMesh(core_axis_name="c", subcore_axis_name="s",
                                    num_cores=NC, num_subcores=NS),
        out_type=jax.ShapeDtypeStruct((NC * N, DP), _BF),
        scratch_types=[
            pltpu.VMEM((EPW,), jnp.int32),
            pltpu.VMEM((EPW,), _F32),
            pltpu.VMEM((CH,), jnp.int32),
            pltpu.VMEM((CH,), jnp.int32),
            pltpu.VMEM((CH, DP), _BF),
            pltpu.VMEM((CH, DP), _BF),
            pltpu.VMEM_SHARED((N, DP), _BF),
            pltpu.SemaphoreType.DMA,
            pltpu.SemaphoreType.DMA,
            pltpu.SemaphoreType.DMA,
            pltpu.SemaphoreType.DMA,
            pltpu.SemaphoreType.DMA,
            pltpu.SemaphoreType.DMA,
        ],
        compiler_params=pltpu.CompilerParams(use_tc_tiling_on_sc=False,
                                             needs_layout_passes=False),
    )
    return fn(xp, zrs, src, dst, w)


def _prep_body(feat, mask, out):
    m = (mask[...] < PROB_ATTR).astype(_F32)
    x = feat[...] * m
    pad = jnp.concatenate(
        [jnp.ones((BR, 1), _F32), jnp.zeros((BR, DP - D - 1), _F32)], axis=1)
    out[...] = jnp.concatenate([x, pad], axis=1).astype(_BF)


def _prep(feat1, mask2):
    return pl.pallas_call(
        _prep_body,
        grid=(GR,),
        in_specs=[
            pl.BlockSpec((BR, D), lambda i: (i, 0)),
            pl.BlockSpec((1, D), lambda i: (0, 0)),
        ],
        out_specs=pl.BlockSpec((BR, DP), lambda i: (i, 0)),
        out_shape=jax.ShapeDtypeStruct((N, DP), _BF),
    )(feat1, mask2)


BS = 1000
GS = N // BS


def _sage_body(accr, xp, ws, wn, b, out, a0v, a1v, s0, s1):
    i = pl.program_id(0)
    c0 = pltpu.async_copy(accr.at[pl.ds(i * BS, BS)], a0v, s0)
    c1 = pltpu.async_copy(accr.at[pl.ds(N + i * BS, BS)], a1v, s1)
    c0.wait()
    c1.wait()
    acc = a0v[...].astype(_F32) + a1v[...].astype(_F32)
    deg = jnp.maximum(acc[:, D:D + 1], 1.0)
    agg = acc[:, :D] / deg
    x = xp[:, :D].astype(_F32)
    h = jnp.dot(x, ws[...], preferred_element_type=_F32)
    h = h + jnp.dot(agg, wn[...], preferred_element_type=_F32)
    h = jnp.maximum(h + b[...], 0.0)
    pad = jnp.concatenate(
        [jnp.ones((BS, 1), _F32), jnp.zeros((BS, DP - D - 1), _F32)], axis=1)
    out[...] = jnp.concatenate([h, pad], axis=1).astype(_BF)


def _sage(acc, xp, ws, wn, b2):
    return pl.pallas_call(
        _sage_body,
        grid=(GS,),
        in_specs=[
            pl.BlockSpec(memory_space=pltpu.HBM),
            pl.BlockSpec((BS, DP), lambda i: (i, 0)),
            pl.BlockSpec((D, D), lambda i: (0, 0)),
            pl.BlockSpec((D, D), lambda i: (0, 0)),
            pl.BlockSpec((1, D), lambda i: (0, 0)),
        ],
        out_specs=pl.BlockSpec((BS, DP), lambda i: (i, 0)),
        out_shape=jax.ShapeDtypeStruct((N, DP), _BF),
        scratch_shapes=[
            pltpu.VMEM((BS, DP), _BF),
            pltpu.VMEM((BS, DP), _BF),
            pltpu.SemaphoreType.DMA,
            pltpu.SemaphoreType.DMA,
        ],
    )(acc, xp, ws, wn, b2)


def _proj_norm(x, wpv, bpv):
    z = jnp.dot(x, wpv, preferred_element_type=_F32) + bpv
    z = jnp.where(z > 0, z, jnp.exp(jnp.minimum(z, 0.0)) - 1.0)
    n = jnp.sqrt(jnp.sum(z * z, axis=1, keepdims=True))
    return ((z / jnp.maximum(n, 1e-12)) * (TEMP ** -0.5)).astype(_BF)


def _proj_h_body(h2p, wp, bp, z1o):
    z1o[...] = _proj_norm(h2p[:, :D].astype(_F32), wp[...], bp[...])


def _proj_e_body(emb, wp, bp, z2o):
    z2o[...] = _proj_norm(emb[...], wp[...], bp[...])


def _proj(x, wp, bp2, from_padded):
    body = _proj_h_body if from_padded else _proj_e_body
    width = DP if from_padded else D
    return pl.pallas_call(
        body,
        grid=(GR,),
        in_specs=[
            pl.BlockSpec((BR, width), lambda i: (i, 0)),
            pl.BlockSpec((D, D), lambda i: (0, 0)),
            pl.BlockSpec((1, D), lambda i: (0, 0)),
        ],
        out_specs=pl.BlockSpec((BR, D), lambda i: (i, 0)),
        out_shape=jax.ShapeDtypeStruct((N, D), _BF),
    )(x, wp, bp2)


_DN = (((1,), (1,)), ((), ()))
_INV_T = 1.0 / TEMP


def _exp_sim(x, y):
    return jnp.exp(lax.dot_general(x, y, _DN, preferred_element_type=_F32))


def _mxu_rowsum(s):
    ones = jnp.ones((BL, 8), _F32)
    return lax.dot_general(s, ones, (((1,), (0,)), ((), ())),
                           preferred_element_type=_F32)[:, :1]


def _vpu_colsum(s):
    return s.sum(axis=0).reshape(1, BL)


def _oh_col(i):
    return (lax.broadcasted_iota(jnp.int32, (1, GL), 1) == i).astype(_F32)


def _oh_row(j):
    return (lax.broadcasted_iota(jnp.int32, (GL, 1), 0) == j).astype(_F32)


def _loss22_body(bi_r, bj_r, r22o, d22o, r22s_, d22s_):
    i = pl.program_id(0)
    j = pl.program_id(1)
    bi = bi_r[...]

    @pl.when((i == 0) & (j == 0))
    def _():
        r22s_[...] = jnp.zeros_like(r22s_)
        d22s_[...] = jnp.zeros_like(d22s_)

    s22 = _exp_sim(bi, bj_r[...])
    r22s_[...] += _mxu_rowsum(s22) * _oh_col(i)

    @pl.when(j == 0)
    def _():
        bi32 = bi.astype(_F32)
        d22 = jnp.exp(jnp.sum(bi32 * bi32, axis=1, keepdims=True))
        d22s_[...] += d22 * _oh_col(i)

    @pl.when((i == GL - 1) & (j == GL - 1))
    def _():
        r22o[...] = r22s_[...]
        d22o[...] = d22s_[...]


def _loss22(z2n):
    return pl.pallas_call(
        _loss22_body,
        grid=(GL, GL),
        in_specs=[
            pl.BlockSpec((BL, D), lambda i, j: (i, 0)),
            pl.BlockSpec((BL, D), lambda i, j: (j, 0)),
        ],
        out_specs=[
            pl.BlockSpec((BL, GL), lambda i, j: (0, 0)),
            pl.BlockSpec((BL, GL), lambda i, j: (0, 0)),
        ],
        out_shape=[
            jax.ShapeDtypeStruct((BL, GL), _F32),
            jax.ShapeDtypeStruct((BL, GL), _F32),
        ],
        scratch_shapes=[pltpu.VMEM((BL, GL), _F32) for _ in range(2)],
    )(z2n, z2n)


def _loss_body(ai_r, bi_r, aj_r, bj_r, r22_r, d22_r, out, r11r, r11c, r12r,
               c12c, d11, ld12):
    i = pl.program_id(0)
    j = pl.program_id(1)
    ai = ai_r[...]
    bi = bi_r[...]
    aj = aj_r[...]
    bj = bj_r[...]

    @pl.when((i == 0) & (j == 0))
    def _():
        r11r[...] = jnp.zeros_like(r11r)
        r11c[...] = jnp.zeros_like(r11c)
        r12r[...] = jnp.zeros_like(r12r)
        c12c[...] = jnp.zeros_like(c12c)
        d11[...] = jnp.zeros_like(d11)
        ld12[...] = jnp.zeros_like(ld12)

    s12 = _exp_sim(ai, bj)
    r12r[...] += _mxu_rowsum(s12) * _oh_col(i)
    c12c[...] += _oh_row(j) * _vpu_colsum(s12)

    @pl.when(j == i)
    def _():
        s11 = _exp_sim(ai, aj)
        r11r[...] += _mxu_rowsum(s11) * _oh_col(i)

    @pl.when(j > i)
    def _():
        s11 = _exp_sim(ai, aj)
        r11r[...] += _mxu_rowsum(s11) * _oh_col(i)
        r11c[...] += _oh_row(j) * _vpu_colsum(s11)

    @pl.when(j == 0)
    def _():
        ai32 = ai.astype(_F32)
        bi32 = bi.astype(_F32)
        oh = _oh_col(i)
        d11[...] += jnp.exp(jnp.sum(ai32 * ai32, axis=1, keepdims=True)) * oh
        ld12[...] += jnp.sum(ai32 * bi32, axis=1, keepdims=True) * oh

    @pl.when((i == GL - 1) & (j == GL - 1))
    def _():
        x1 = r11r[...] + r11c[...].T - d11[...] + r12r[...]
        x2 = r22_r[...] - d22_r[...] + c12c[...].T
        l1 = jnp.log(x1) - ld12[...]
        l2 = jnp.log(x2) - ld12[...]
        out[0, 0] = jnp.sum(l1 + l2) / (2.0 * N)


def _loss(z1n, z2n, r22, d22):
    return pl.pallas_call(
        _loss_body,
        grid=(GL, GL),
        in_specs=[
            pl.BlockSpec((BL, D), lambda i, j: (i, 0)),
            pl.BlockSpec((BL, D), lambda i, j: (i, 0)),
            pl.BlockSpec((BL, D), lambda i, j: (j, 0)),
            pl.BlockSpec((BL, D), lambda i, j: (j, 0)),
            pl.BlockSpec((BL, GL), lambda i, j: (0, 0)),
            pl.BlockSpec((BL, GL), lambda i, j: (0, 0)),
        ],
        out_specs=pl.BlockSpec((1, 1), lambda i, j: (0, 0),
                               memory_space=pltpu.SMEM),
        out_shape=jax.ShapeDtypeStruct((1, 1), _F32),
        scratch_shapes=[
            pltpu.VMEM((BL, GL), _F32),
            pltpu.VMEM((GL, BL), _F32),
            pltpu.VMEM((BL, GL), _F32),
            pltpu.VMEM((GL, BL), _F32),
            pltpu.VMEM((BL, GL), _F32),
            pltpu.VMEM((BL, GL), _F32),
        ],
    )(z1n, z2n, z1n, z2n, r22, d22)


def kernel(feat1, embds, edge_index, edge_weight, mask_rand,
           W_self0, W_neigh0, b0, W_self1, W_neigh1, b1, W_proj, b_proj):
    ei = edge_index.astype(jnp.int32)
    src = ei[0]
    dst = ei[1]
    mask2 = mask_rand.reshape(1, D)
    b0r = b0.reshape(1, D)
    b1r = b1.reshape(1, D)
    bpr = b_proj.reshape(1, D)
    zrs = jnp.zeros((RPT, DP), _BF)

    z2n = _proj(embds, W_proj, bpr, from_padded=False)
    r22, d22 = _loss22(z2n)

    xp = _prep(feat1, mask2)
    acc1 = _sc_agg(xp, zrs, src, dst, edge_weight)
    h1p = _sage(acc1, xp, W_self0, W_neigh0, b0r)
    acc2 = _sc_agg(h1p, zrs, src, dst, edge_weight)
    h2p = _sage(acc2, h1p, W_self1, W_neigh1, b1r)
    z1n = _proj(h2p, W_proj, bpr, from_padded=True)
    out = _loss(z1n, z2n, r22, d22)
    return out.reshape(())


# speedup vs baseline: 1.0357x; 1.0176x over previous
import functools

import jax
import jax.numpy as jnp
from jax import lax
from jax.experimental import pallas as pl
from jax.experimental.pallas import tpu as pltpu
from jax.experimental.pallas import tpu_sc as plsc

N = 10000
D = 128
DP = 160
E = 320000
NC, NS = 2, 16
NW = NC * NS
EPW = E // NW
CH = 80
NCHUNK = EPW // CH
RPT = N // NS
PROB_ATTR = 0.5
TEMP = 0.5

BR = 400
GR = N // BR
BL = 2000
GL = N // BL

_F32 = jnp.float32
_BF = jnp.bfloat16


def _sc_agg_body(xp, zrs, ei, w, out,
                 srcv, wv, dstage0, dstage1, rows0, rows1, acc,
                 sem0, sem1, semd0, semd1, sems0, sems1):
    c = lax.axis_index("c")
    s = lax.axis_index("s")
    base = (c * NS + s) * EPW
    pltpu.sync_copy(ei.at[0, pl.ds(base, EPW)], srcv)
    pltpu.sync_copy(w.at[pl.ds(base, EPW)], wv)
    pltpu.sync_copy(zrs, acc.at[pl.ds(s * RPT, RPT)])
    plsc.subcore_barrier()

    rows = (rows0, rows1)
    sems = (sem0, sem1)
    dstage = (dstage0, dstage1)
    semsd = (semd0, semd1)
    semss = (sems0, sems1)

    def scale(i, rbuf):
        @plsc.parallel_loop(0, CH, unroll=4)
        def _(r):
            wr = plsc.load_gather(wv, [jnp.full((16,), i * CH, jnp.int32) + r])
            wrb = plsc.pack(wr, wr, format=plsc.PackFormat.INTERLEAVED)
            for k in range(D // 32):
                rbuf[r, pl.ds(k * 32, 32)] = rbuf[r, pl.ds(k * 32, 32)] * wrb

    def fetch(i, b):
        pltpu.async_copy(xp.at[srcv.at[pl.ds(i * CH, CH)]], rows[b], sems[b])
        pltpu.async_copy(ei.at[1, pl.ds(base + i * CH, CH)], dstage[b],
                         semsd[b])

    def await_chunk(i, b):
        pltpu.make_async_copy(xp.at[srcv.at[pl.ds(i * CH, CH)]],
                              rows[b], sems[b]).wait()
        pltpu.make_async_copy(ei.at[1, pl.ds(base + i * CH, CH)], dstage[b],
                              semsd[b]).wait()

    def await_scatter(b):
        pltpu.make_async_copy(rows[b], acc.at[dstage[b]], semss[b]).wait()

    fetch(0, 0)

    def pair(p, carry):
        for b in range(2):
            i = p * 2 + b
            await_chunk(i, b)

            @pl.when(i > 0)
            def _():
                await_scatter(1 - b)

            fetch(i + 1, 1 - b)
            scale(i, rows[b])
            pltpu.async_copy(rows[b], acc.at[dstage[b]], semss[b], add=True)
        return carry

    lax.fori_loop(0, NCHUNK // 2, pair, 0)
    last = NCHUNK - 1
    await_chunk(last, 0)
    await_scatter(1)
    scale(last, rows0)
    pltpu.async_copy(rows0, acc.at[dstage0], semss[0], add=True)
    await_scatter(0)
    plsc.subcore_barrier()
    pltpu.sync_copy(acc.at[pl.ds(s * RPT, RPT)],
                    out.at[pl.ds(c * N + s * RPT, RPT)])


def _sc_agg(xp, zrs, ei, w):
    fn = pl.kernel(
        _sc_agg_body,
        mesh=plsc.VectorSubcoreMesh(core_axis_name="c", subcore_axis_name="s",
                                    num_cores=NC, num_subcores=NS),
        out_type=jax.ShapeDtypeStruct((NC * N, DP), _BF),
        scratch_types=[
            pltpu.VMEM((EPW,), jnp.int32),
            pltpu.VMEM((EPW,), _F32),
            pltpu.VMEM((CH,), jnp.int32),
            pltpu.VMEM((CH,), jnp.int32),
            pltpu.VMEM((CH, DP), _BF),
            pltpu.VMEM((CH, DP), _BF),
            pltpu.VMEM_SHARED((N, DP), _BF),
            pltpu.SemaphoreType.DMA,
            pltpu.SemaphoreType.DMA,
            pltpu.SemaphoreType.DMA,
            pltpu.SemaphoreType.DMA,
            pltpu.SemaphoreType.DMA,
            pltpu.SemaphoreType.DMA,
        ],
        compiler_params=pltpu.CompilerParams(use_tc_tiling_on_sc=False,
                                             needs_layout_passes=False),
    )
    return fn(xp, zrs, ei, w)


def _prep_body(feat, mask, out):
    m = (mask[...] < PROB_ATTR).astype(_F32)
    x = feat[...] * m
    pad = jnp.concatenate(
        [jnp.ones((BR, 1), _F32), jnp.zeros((BR, DP - D - 1), _F32)], axis=1)
    out[...] = jnp.concatenate([x, pad], axis=1).astype(_BF)


def _prep(feat1, mask2):
    return pl.pallas_call(
        _prep_body,
        grid=(GR,),
        in_specs=[
            pl.BlockSpec((BR, D), lambda i: (i, 0)),
            pl.BlockSpec((1, D), lambda i: (0, 0)),
        ],
        out_specs=pl.BlockSpec((BR, DP), lambda i: (i, 0)),
        out_shape=jax.ShapeDtypeStruct((N, DP), _BF),
    )(feat1, mask2)


BS = 1000
GS = N // BS


def _sage_body(accr, xp, ws, wn, b, out, a0v, a1v, s0, s1):
    i = pl.program_id(0)
    c0 = pltpu.async_copy(accr.at[pl.ds(i * BS, BS)], a0v, s0)
    c1 = pltpu.async_copy(accr.at[pl.ds(N + i * BS, BS)], a1v, s1)
    c0.wait()
    c1.wait()
    acc = a0v[...].astype(_F32) + a1v[...].astype(_F32)
    deg = jnp.maximum(acc[:, D:D + 1], 1.0)
    agg = acc[:, :D] / deg
    x = xp[:, :D].astype(_F32)
    h = jnp.dot(x, ws[...], preferred_element_type=_F32)
    h = h + jnp.dot(agg, wn[...], preferred_element_type=_F32)
    h = jnp.maximum(h + b[...], 0.0)
    pad = jnp.concatenate(
        [jnp.ones((BS, 1), _F32), jnp.zeros((BS, DP - D - 1), _F32)], axis=1)
    out[...] = jnp.concatenate([h, pad], axis=1).astype(_BF)


def _sage(acc, xp, ws, wn, b2):
    return pl.pallas_call(
        _sage_body,
        grid=(GS,),
        in_specs=[
            pl.BlockSpec(memory_space=pltpu.HBM),
            pl.BlockSpec((BS, DP), lambda i: (i, 0)),
            pl.BlockSpec((D, D), lambda i: (0, 0)),
            pl.BlockSpec((D, D), lambda i: (0, 0)),
            pl.BlockSpec((1, D), lambda i: (0, 0)),
        ],
        out_specs=pl.BlockSpec((BS, DP), lambda i: (i, 0)),
        out_shape=jax.ShapeDtypeStruct((N, DP), _BF),
        scratch_shapes=[
            pltpu.VMEM((BS, DP), _BF),
            pltpu.VMEM((BS, DP), _BF),
            pltpu.SemaphoreType.DMA,
            pltpu.SemaphoreType.DMA,
        ],
    )(acc, xp, ws, wn, b2)


def _proj_norm(x, wpv, bpv):
    z = jnp.dot(x, wpv, preferred_element_type=_F32) + bpv
    z = jnp.where(z > 0, z, jnp.exp(jnp.minimum(z, 0.0)) - 1.0)
    n = jnp.sqrt(jnp.sum(z * z, axis=1, keepdims=True))
    return ((z / jnp.maximum(n, 1e-12)) * (TEMP ** -0.5)).astype(_BF)


def _proj_h_body(h2p, wp, bp, z1o):
    z1o[...] = _proj_norm(h2p[:, :D].astype(_F32), wp[...], bp[...])


def _proj_e_body(emb, wp, bp, z2o):
    z2o[...] = _proj_norm(emb[...], wp[...], bp[...])


def _proj(x, wp, bp2, from_padded):
    body = _proj_h_body if from_padded else _proj_e_body
    width = DP if from_padded else D
    return pl.pallas_call(
        body,
        grid=(GR,),
        in_specs=[
            pl.BlockSpec((BR, width), lambda i: (i, 0)),
            pl.BlockSpec((D, D), lambda i: (0, 0)),
            pl.BlockSpec((1, D), lambda i: (0, 0)),
        ],
        out_specs=pl.BlockSpec((BR, D), lambda i: (i, 0)),
        out_shape=jax.ShapeDtypeStruct((N, D), _BF),
    )(x, wp, bp2)


_DN = (((1,), (1,)), ((), ()))
_INV_T = 1.0 / TEMP


def _exp_sim(x, y):
    return jnp.exp(lax.dot_general(x, y, _DN, preferred_element_type=_F32))


def _mxu_rowsum(s):
    ones = jnp.ones((BL, 8), _F32)
    return lax.dot_general(s, ones, (((1,), (0,)), ((), ())),
                           preferred_element_type=_F32)[:, :1]


def _vpu_colsum(s):
    return s.sum(axis=0).reshape(1, BL)


def _oh_col(i):
    return (lax.broadcasted_iota(jnp.int32, (1, GL), 1) == i).astype(_F32)


def _oh_row(j):
    return (lax.broadcasted_iota(jnp.int32, (GL, 1), 0) == j).astype(_F32)


def _loss22_body(bi_r, bj_r, r22o, d22o, r22s_, d22s_):
    i = pl.program_id(0)
    j = pl.program_id(1)
    bi = bi_r[...]

    @pl.when((i == 0) & (j == 0))
    def _():
        r22s_[...] = jnp.zeros_like(r22s_)
        d22s_[...] = jnp.zeros_like(d22s_)

    s22 = _exp_sim(bi, bj_r[...])
    r22s_[...] += _mxu_rowsum(s22) * _oh_col(i)

    @pl.when(j == 0)
    def _():
        bi32 = bi.astype(_F32)
        d22 = jnp.exp(jnp.sum(bi32 * bi32, axis=1, keepdims=True))
        d22s_[...] += d22 * _oh_col(i)

    @pl.when((i == GL - 1) & (j == GL - 1))
    def _():
        r22o[...] = r22s_[...]
        d22o[...] = d22s_[...]


def _loss22(z2n):
    return pl.pallas_call(
        _loss22_body,
        grid=(GL, GL),
        in_specs=[
            pl.BlockSpec((BL, D), lambda i, j: (i, 0)),
            pl.BlockSpec((BL, D), lambda i, j: (j, 0)),
        ],
        out_specs=[
            pl.BlockSpec((BL, GL), lambda i, j: (0, 0)),
            pl.BlockSpec((BL, GL), lambda i, j: (0, 0)),
        ],
        out_shape=[
            jax.ShapeDtypeStruct((BL, GL), _F32),
            jax.ShapeDtypeStruct((BL, GL), _F32),
        ],
        scratch_shapes=[pltpu.VMEM((BL, GL), _F32) for _ in range(2)],
    )(z2n, z2n)


def _loss_body(ai_r, bi_r, aj_r, bj_r, r22_r, d22_r, out, r11r, r11c, r12r,
               c12c, d11, ld12):
    i = pl.program_id(0)
    j = pl.program_id(1)
    ai = ai_r[...]
    bi = bi_r[...]
    aj = aj_r[...]
    bj = bj_r[...]

    @pl.when((i == 0) & (j == 0))
    def _():
        r11r[...] = jnp.zeros_like(r11r)
        r11c[...] = jnp.zeros_like(r11c)
        r12r[...] = jnp.zeros_like(r12r)
        c12c[...] = jnp.zeros_like(c12c)
        d11[...] = jnp.zeros_like(d11)
        ld12[...] = jnp.zeros_like(ld12)

    s12 = _exp_sim(ai, bj)
    r12r[...] += _mxu_rowsum(s12) * _oh_col(i)
    c12c[...] += _oh_row(j) * _vpu_colsum(s12)

    @pl.when(j == i)
    def _():
        s11 = _exp_sim(ai, aj)
        r11r[...] += _mxu_rowsum(s11) * _oh_col(i)

    @pl.when(j > i)
    def _():
        s11 = _exp_sim(ai, aj)
        r11r[...] += _mxu_rowsum(s11) * _oh_col(i)
        r11c[...] += _oh_row(j) * _vpu_colsum(s11)

    @pl.when(j == 0)
    def _():
        ai32 = ai.astype(_F32)
        bi32 = bi.astype(_F32)
        oh = _oh_col(i)
        d11[...] += jnp.exp(jnp.sum(ai32 * ai32, axis=1, keepdims=True)) * oh
        ld12[...] += jnp.sum(ai32 * bi32, axis=1, keepdims=True) * oh

    @pl.when((i == GL - 1) & (j == GL - 1))
    def _():
        x1 = r11r[...] + r11c[...].T - d11[...] + r12r[...]
        x2 = r22_r[...] - d22_r[...] + c12c[...].T
        l1 = jnp.log(x1) - ld12[...]
        l2 = jnp.log(x2) - ld12[...]
        out[0, 0] = jnp.sum(l1 + l2) / (2.0 * N)


def _loss(z1n, z2n, r22, d22):
    return pl.pallas_call(
        _loss_body,
        grid=(GL, GL),
        in_specs=[
            pl.BlockSpec((BL, D), lambda i, j: (i, 0)),
            pl.BlockSpec((BL, D), lambda i, j: (i, 0)),
            pl.BlockSpec((BL, D), lambda i, j: (j, 0)),
            pl.BlockSpec((BL, D), lambda i, j: (j, 0)),
            pl.BlockSpec((BL, GL), lambda i, j: (0, 0)),
            pl.BlockSpec((BL, GL), lambda i, j: (0, 0)),
        ],
        out_specs=pl.BlockSpec((1, 1), lambda i, j: (0, 0),
                               memory_space=pltpu.SMEM),
        out_shape=jax.ShapeDtypeStruct((1, 1), _F32),
        scratch_shapes=[
            pltpu.VMEM((BL, GL), _F32),
            pltpu.VMEM((GL, BL), _F32),
            pltpu.VMEM((BL, GL), _F32),
            pltpu.VMEM((GL, BL), _F32),
            pltpu.VMEM((BL, GL), _F32),
            pltpu.VMEM((BL, GL), _F32),
        ],
    )(z1n, z2n, z1n, z2n, r22, d22)


def kernel(feat1, embds, edge_index, edge_weight, mask_rand,
           W_self0, W_neigh0, b0, W_self1, W_neigh1, b1, W_proj, b_proj):
    ei = edge_index.astype(jnp.int32)
    mask2 = mask_rand.reshape(1, D)
    b0r = b0.reshape(1, D)
    b1r = b1.reshape(1, D)
    bpr = b_proj.reshape(1, D)
    zrs = jnp.zeros((RPT, DP), _BF)

    z2n = _proj(embds, W_proj, bpr, from_padded=False)
    r22, d22 = _loss22(z2n)

    xp = _prep(feat1, mask2)
    acc1 = _sc_agg(xp, zrs, ei, edge_weight)
    h1p = _sage(acc1, xp, W_self0, W_neigh0, b0r)
    acc2 = _sc_agg(h1p, zrs, ei, edge_weight)
    h2p = _sage(acc2, h1p, W_self1, W_neigh1, b1r)
    z1n = _proj(h2p, W_proj, bpr, from_padded=True)
    out = _loss(z1n, z2n, r22, d22)
    return out.reshape(())


# speedup vs baseline: 1.0564x; 1.0200x over previous
import functools

import jax
import jax.numpy as jnp
from jax import lax
from jax.experimental import pallas as pl
from jax.experimental.pallas import tpu as pltpu
from jax.experimental.pallas import tpu_sc as plsc

N = 10000
D = 128
DP = 160
E = 320000
NC, NS = 2, 16
NW = NC * NS
EPW = E // NW
CH = 80
NCHUNK = EPW // CH
RPT = N // NS
PROB_ATTR = 0.5
TEMP = 0.5

BR = 400
GR = N // BR
BL = 2000
GL = N // BL

_F32 = jnp.float32
_BF = jnp.bfloat16


def _sc_agg_body(xp, zrs, ei, w, out,
                 srcv, wv, dstage0, dstage1, rows0, rows1, acc,
                 sem0, sem1, semd0, semd1, sems0, sems1):
    c = lax.axis_index("c")
    s = lax.axis_index("s")
    base = (c * NS + s) * EPW
    pltpu.sync_copy(ei.at[0, pl.ds(base, EPW)], srcv)
    pltpu.sync_copy(w.at[pl.ds(base, EPW)], wv)
    pltpu.sync_copy(zrs, acc.at[pl.ds(s * RPT, RPT)])
    plsc.subcore_barrier()

    rows = (rows0, rows1)
    sems = (sem0, sem1)
    dstage = (dstage0, dstage1)
    semsd = (semd0, semd1)
    semss = (sems0, sems1)

    def scale(i, rbuf):
        @plsc.parallel_loop(0, CH, unroll=4)
        def _(r):
            wr = plsc.load_gather(wv, [jnp.full((16,), i * CH, jnp.int32) + r])
            wrb = plsc.pack(wr, wr, format=plsc.PackFormat.INTERLEAVED)
            for k in range(D // 32):
                rbuf[r, pl.ds(k * 32, 32)] = rbuf[r, pl.ds(k * 32, 32)] * wrb

    def fetch(i, b):
        pltpu.async_copy(xp.at[srcv.at[pl.ds(i * CH, CH)]], rows[b], sems[b])
        pltpu.async_copy(ei.at[1, pl.ds(base + i * CH, CH)], dstage[b],
                         semsd[b])

    def await_chunk(i, b):
        pltpu.make_async_copy(xp.at[srcv.at[pl.ds(i * CH, CH)]],
                              rows[b], sems[b]).wait()
        pltpu.make_async_copy(ei.at[1, pl.ds(base + i * CH, CH)], dstage[b],
                              semsd[b]).wait()

    def await_scatter(b):
        pltpu.make_async_copy(rows[b], acc.at[dstage[b]], semss[b]).wait()

    fetch(0, 0)

    def pair(p, carry):
        for b in range(2):
            i = p * 2 + b
            await_chunk(i, b)

            @pl.when(i > 0)
            def _():
                await_scatter(1 - b)

            fetch(i + 1, 1 - b)
            scale(i, rows[b])
            pltpu.async_copy(rows[b], acc.at[dstage[b]], semss[b], add=True)
        return carry

    lax.fori_loop(0, NCHUNK // 2, pair, 0)
    last = NCHUNK - 1
    await_chunk(last, 0)
    await_scatter(1)
    scale(last, rows0)
    pltpu.async_copy(rows0, acc.at[dstage0], semss[0], add=True)
    await_scatter(0)
    plsc.subcore_barrier()
    pltpu.sync_copy(acc.at[pl.ds(s * RPT, RPT)],
                    out.at[pl.ds(c * N + s * RPT, RPT)])


def _sc_agg(xp, zrs, ei, w):
    fn = pl.kernel(
        _sc_agg_body,
        mesh=plsc.VectorSubcoreMesh(core_axis_name="c", subcore_axis_name="s",
                                    num_cores=NC, num_subcores=NS),
        out_type=jax.ShapeDtypeStruct((NC * N, DP), _BF),
        scratch_types=[
            pltpu.VMEM((EPW,), jnp.int32),
            pltpu.VMEM((EPW,), _F32),
            pltpu.VMEM((CH,), jnp.int32),
            pltpu.VMEM((CH,), jnp.int32),
            pltpu.VMEM((CH, DP), _BF),
            pltpu.VMEM((CH, DP), _BF),
            pltpu.VMEM_SHARED((N, DP), _BF),
            pltpu.SemaphoreType.DMA,
            pltpu.SemaphoreType.DMA,
            pltpu.SemaphoreType.DMA,
            pltpu.SemaphoreType.DMA,
            pltpu.SemaphoreType.DMA,
            pltpu.SemaphoreType.DMA,
        ],
        compiler_params=pltpu.CompilerParams(use_tc_tiling_on_sc=False,
                                             needs_layout_passes=False),
    )
    return fn(xp, zrs, ei, w)


def _prep_body(feat, mask, out):
    m = (mask[...] < PROB_ATTR).astype(_F32)
    x = feat[...] * m
    pad = jnp.concatenate(
        [jnp.ones((BR, 1), _F32), jnp.zeros((BR, DP - D - 1), _F32)], axis=1)
    out[...] = jnp.concatenate([x, pad], axis=1).astype(_BF)


def _prep(feat1, mask2):
    return pl.pallas_call(
        _prep_body,
        grid=(GR,),
        in_specs=[
            pl.BlockSpec((BR, D), lambda i: (i, 0)),
            pl.BlockSpec((1, D), lambda i: (0, 0)),
        ],
        out_specs=pl.BlockSpec((BR, DP), lambda i: (i, 0)),
        out_shape=jax.ShapeDtypeStruct((N, DP), _BF),
    )(feat1, mask2)


BS = 1000
GS = N // BS


def _sage_body(accr, xp, ws, wn, b, out, a0v, a1v, s0, s1):
    i = pl.program_id(0)
    c0 = pltpu.async_copy(accr.at[pl.ds(i * BS, BS)], a0v, s0)
    c1 = pltpu.async_copy(accr.at[pl.ds(N + i * BS, BS)], a1v, s1)
    c0.wait()
    c1.wait()
    acc = a0v[...].astype(_F32) + a1v[...].astype(_F32)
    deg = jnp.maximum(acc[:, D:D + 1], 1.0)
    agg = acc[:, :D] / deg
    x = xp[:, :D].astype(_F32)
    h = jnp.dot(x, ws[...], preferred_element_type=_F32)
    h = h + jnp.dot(agg, wn[...], preferred_element_type=_F32)
    h = jnp.maximum(h + b[...], 0.0)
    pad = jnp.concatenate(
        [jnp.ones((BS, 1), _F32), jnp.zeros((BS, DP - D - 1), _F32)], axis=1)
    out[...] = jnp.concatenate([h, pad], axis=1).astype(_BF)


def _sage(acc, xp, ws, wn, b2):
    return pl.pallas_call(
        _sage_body,
        grid=(GS,),
        in_specs=[
            pl.BlockSpec(memory_space=pltpu.HBM),
            pl.BlockSpec((BS, DP), lambda i: (i, 0)),
            pl.BlockSpec((D, D), lambda i: (0, 0)),
            pl.BlockSpec((D, D), lambda i: (0, 0)),
            pl.BlockSpec((1, D), lambda i: (0, 0)),
        ],
        out_specs=pl.BlockSpec((BS, DP), lambda i: (i, 0)),
        out_shape=jax.ShapeDtypeStruct((N, DP), _BF),
        scratch_shapes=[
            pltpu.VMEM((BS, DP), _BF),
            pltpu.VMEM((BS, DP), _BF),
            pltpu.SemaphoreType.DMA,
            pltpu.SemaphoreType.DMA,
        ],
    )(acc, xp, ws, wn, b2)


def _proj_norm(x, wpv, bpv):
    z = jnp.dot(x, wpv, preferred_element_type=_F32) + bpv
    z = jnp.where(z > 0, z, jnp.exp(jnp.minimum(z, 0.0)) - 1.0)
    n = jnp.sqrt(jnp.sum(z * z, axis=1, keepdims=True))
    return ((z / jnp.maximum(n, 1e-12)) * (TEMP ** -0.5)).astype(_BF)


def _sage_proj_body(accr, xp, ws, wn, b, wp, bp, z1o, a0v, a1v, s0, s1):
    i = pl.program_id(0)
    c0 = pltpu.async_copy(accr.at[pl.ds(i * BS, BS)], a0v, s0)
    c1 = pltpu.async_copy(accr.at[pl.ds(N + i * BS, BS)], a1v, s1)
    c0.wait()
    c1.wait()
    acc = a0v[...].astype(_F32) + a1v[...].astype(_F32)
    deg = jnp.maximum(acc[:, D:D + 1], 1.0)
    agg = acc[:, :D] / deg
    x = xp[:, :D].astype(_F32)
    h = jnp.dot(x, ws[...], preferred_element_type=_F32)
    h = h + jnp.dot(agg, wn[...], preferred_element_type=_F32)
    h = jnp.maximum(h + b[...], 0.0)
    z1o[...] = _proj_norm(h, wp[...], bp[...])


def _sage_proj(acc, xp, ws, wn, b2, wp, bp2):
    return pl.pallas_call(
        _sage_proj_body,
        grid=(GS,),
        in_specs=[
            pl.BlockSpec(memory_space=pltpu.HBM),
            pl.BlockSpec((BS, DP), lambda i: (i, 0)),
            pl.BlockSpec((D, D), lambda i: (0, 0)),
            pl.BlockSpec((D, D), lambda i: (0, 0)),
            pl.BlockSpec((1, D), lambda i: (0, 0)),
            pl.BlockSpec((D, D), lambda i: (0, 0)),
            pl.BlockSpec((1, D), lambda i: (0, 0)),
        ],
        out_specs=pl.BlockSpec((BS, D), lambda i: (i, 0)),
        out_shape=jax.ShapeDtypeStruct((N, D), _BF),
        scratch_shapes=[
            pltpu.VMEM((BS, DP), _BF),
            pltpu.VMEM((BS, DP), _BF),
            pltpu.SemaphoreType.DMA,
            pltpu.SemaphoreType.DMA,
        ],
    )(acc, xp, ws, wn, b2, wp, bp2)


def _proj_e_body(emb, wp, bp, z2o):
    z2o[...] = _proj_norm(emb[...], wp[...], bp[...])


def _proj(x, wp, bp2):
    return pl.pallas_call(
        _proj_e_body,
        grid=(GR,),
        in_specs=[
            pl.BlockSpec((BR, D), lambda i: (i, 0)),
            pl.BlockSpec((D, D), lambda i: (0, 0)),
            pl.BlockSpec((1, D), lambda i: (0, 0)),
        ],
        out_specs=pl.BlockSpec((BR, D), lambda i: (i, 0)),
        out_shape=jax.ShapeDtypeStruct((N, D), _BF),
    )(x, wp, bp2)


_DN = (((1,), (1,)), ((), ()))
_INV_T = 1.0 / TEMP


def _exp_sim(x, y):
    return jnp.exp(lax.dot_general(x, y, _DN, preferred_element_type=_F32))


def _mxu_rowsum(s):
    ones = jnp.ones((BL, 8), _F32)
    return lax.dot_general(s, ones, (((1,), (0,)), ((), ())),
                           preferred_element_type=_F32)[:, :1]


def _vpu_colsum(s):
    return s.sum(axis=0).reshape(1, BL)


def _oh_col(i):
    return (lax.broadcasted_iota(jnp.int32, (1, GL), 1) == i).astype(_F32)


def _oh_row(j):
    return (lax.broadcasted_iota(jnp.int32, (GL, 1), 0) == j).astype(_F32)


def _loss22_body(bi_r, bj_r, r22o, d22o, r22s_, d22s_):
    i = pl.program_id(0)
    j = pl.program_id(1)
    bi = bi_r[...]

    @pl.when((i == 0) & (j == 0))
    def _():
        r22s_[...] = jnp.zeros_like(r22s_)
        d22s_[...] = jnp.zeros_like(d22s_)

    s22 = _exp_sim(bi, bj_r[...])
    r22s_[...] += _mxu_rowsum(s22) * _oh_col(i)

    @pl.when(j == 0)
    def _():
        bi32 = bi.astype(_F32)
        d22 = jnp.exp(jnp.sum(bi32 * bi32, axis=1, keepdims=True))
        d22s_[...] += d22 * _oh_col(i)

    @pl.when((i == GL - 1) & (j == GL - 1))
    def _():
        r22o[...] = r22s_[...]
        d22o[...] = d22s_[...]


def _loss22(z2n):
    return pl.pallas_call(
        _loss22_body,
        grid=(GL, GL),
        in_specs=[
            pl.BlockSpec((BL, D), lambda i, j: (i, 0)),
            pl.BlockSpec((BL, D), lambda i, j: (j, 0)),
        ],
        out_specs=[
            pl.BlockSpec((BL, GL), lambda i, j: (0, 0)),
            pl.BlockSpec((BL, GL), lambda i, j: (0, 0)),
        ],
        out_shape=[
            jax.ShapeDtypeStruct((BL, GL), _F32),
            jax.ShapeDtypeStruct((BL, GL), _F32),
        ],
        scratch_shapes=[pltpu.VMEM((BL, GL), _F32) for _ in range(2)],
    )(z2n, z2n)


def _loss_body(ai_r, bi_r, aj_r, bj_r, r22_r, d22_r, out, r11r, r11c, r12r,
               c12c, d11, ld12):
    i = pl.program_id(0)
    j = pl.program_id(1)
    ai = ai_r[...]
    bi = bi_r[...]
    aj = aj_r[...]
    bj = bj_r[...]

    @pl.when((i == 0) & (j == 0))
    def _():
        r11r[...] = jnp.zeros_like(r11r)
        r11c[...] = jnp.zeros_like(r11c)
        r12r[...] = jnp.zeros_like(r12r)
        c12c[...] = jnp.zeros_like(c12c)
        d11[...] = jnp.zeros_like(d11)
        ld12[...] = jnp.zeros_like(ld12)

    s12 = _exp_sim(ai, bj)
    r12r[...] += _mxu_rowsum(s12) * _oh_col(i)
    c12c[...] += _oh_row(j) * _vpu_colsum(s12)

    @pl.when(j == i)
    def _():
        s11 = _exp_sim(ai, aj)
        r11r[...] += _mxu_rowsum(s11) * _oh_col(i)

    @pl.when(j > i)
    def _():
        s11 = _exp_sim(ai, aj)
        r11r[...] += _mxu_rowsum(s11) * _oh_col(i)
        r11c[...] += _oh_row(j) * _vpu_colsum(s11)

    @pl.when(j == 0)
    def _():
        ai32 = ai.astype(_F32)
        bi32 = bi.astype(_F32)
        oh = _oh_col(i)
        d11[...] += jnp.exp(jnp.sum(ai32 * ai32, axis=1, keepdims=True)) * oh
        ld12[...] += jnp.sum(ai32 * bi32, axis=1, keepdims=True) * oh

    @pl.when((i == GL - 1) & (j == GL - 1))
    def _():
        x1 = r11r[...] + r11c[...].T - d11[...] + r12r[...]
        x2 = r22_r[...] - d22_r[...] + c12c[...].T
        l1 = jnp.log(x1) - ld12[...]
        l2 = jnp.log(x2) - ld12[...]
        out[0, 0] = jnp.sum(l1 + l2) / (2.0 * N)


def _loss(z1n, z2n, r22, d22):
    return pl.pallas_call(
        _loss_body,
        grid=(GL, GL),
        in_specs=[
            pl.BlockSpec((BL, D), lambda i, j: (i, 0)),
            pl.BlockSpec((BL, D), lambda i, j: (i, 0)),
            pl.BlockSpec((BL, D), lambda i, j: (j, 0)),
            pl.BlockSpec((BL, D), lambda i, j: (j, 0)),
            pl.BlockSpec((BL, GL), lambda i, j: (0, 0)),
            pl.BlockSpec((BL, GL), lambda i, j: (0, 0)),
        ],
        out_specs=pl.BlockSpec((1, 1), lambda i, j: (0, 0),
                               memory_space=pltpu.SMEM),
        out_shape=jax.ShapeDtypeStruct((1, 1), _F32),
        scratch_shapes=[
            pltpu.VMEM((BL, GL), _F32),
            pltpu.VMEM((GL, BL), _F32),
            pltpu.VMEM((BL, GL), _F32),
            pltpu.VMEM((GL, BL), _F32),
            pltpu.VMEM((BL, GL), _F32),
            pltpu.VMEM((BL, GL), _F32),
        ],
    )(z1n, z2n, z1n, z2n, r22, d22)


def kernel(feat1, embds, edge_index, edge_weight, mask_rand,
           W_self0, W_neigh0, b0, W_self1, W_neigh1, b1, W_proj, b_proj):
    ei = edge_index.astype(jnp.int32)
    mask2 = mask_rand.reshape(1, D)
    b0r = b0.reshape(1, D)
    b1r = b1.reshape(1, D)
    bpr = b_proj.reshape(1, D)
    zrs = jnp.zeros((RPT, DP), _BF)

    z2n = _proj(embds, W_proj, bpr)
    r22, d22 = _loss22(z2n)

    xp = _prep(feat1, mask2)
    acc1 = _sc_agg(xp, zrs, ei, edge_weight)
    h1p = _sage(acc1, xp, W_self0, W_neigh0, b0r)
    acc2 = _sc_agg(h1p, zrs, ei, edge_weight)
    z1n = _sage_proj(acc2, h1p, W_self1, W_neigh1, b1r, W_proj, bpr)
    out = _loss(z1n, z2n, r22, d22)
    return out.reshape(())
